# Initial kernel scaffold; baseline (speedup 1.0000x reference)
import jax
import jax.numpy as jnp
from jax.experimental import pallas as pl


def kernel(fea, W_up, b_up, W_cls, b_cls, target_points_num):
    raise NotImplementedError("write your pallas kernel here")


# speedup vs baseline: 1.1275x; 1.1275x over previous
import functools

import jax
import jax.numpy as jnp
from jax import lax
from jax.experimental import pallas as pl
from jax.experimental.pallas import tpu as pltpu
from jax.experimental.pallas import tpu_sc as plsc

TGT = 131072
NC, NS = 2, 16
NW = NC * NS
GBLK = 128


def _key_from_f32(x_i32):
    return x_i32 ^ (lax.shift_right_arithmetic(x_i32, 31) & jnp.int32(0x7FFFFFFF))


def _k1_body(fea_ref, wup_ref, bup_ref, wcls_ref, bcls_ref, f_ref, key_ref):
    x = fea_ref[...]
    f = jnp.maximum(
        jnp.dot(x, wup_ref[...], preferred_element_type=jnp.float32)
        + bup_ref[...], 0.0)
    f_ref[...] = f
    pred = (jnp.dot(f, wcls_ref[...], preferred_element_type=jnp.float32)
            + bcls_ref[...])
    key_ref[...] = _key_from_f32(lax.bitcast_convert_type(pred, jnp.int32))


def _k2_body(key_ref, meta_ref, *, n, chunk_rows):
    key = key_ref[...]
    k0 = n - TGT

    def step(i, res):
        cand = res + lax.shift_left(jnp.int32(1), 31 - i)
        cnt = jnp.sum((key < cand).astype(jnp.int32))
        return jnp.where(cnt >= k0, res, cand)

    tkey = lax.fori_loop(0, 32, step, jnp.int32(-2147483647 - 1))

    keepf = (key > tkey).astype(jnp.float32)
    rowsum = jnp.sum(keepf, axis=1, keepdims=True)
    rows = key.shape[0]
    r_idx = lax.broadcasted_iota(jnp.int32, (NW, rows), 1)
    c_idx = lax.broadcasted_iota(jnp.int32, (NW, rows), 0)
    sel = (r_idx // chunk_rows == c_idx).astype(jnp.float32)
    counts = jnp.dot(sel, rowsum, preferred_element_type=jnp.float32,
                     precision=lax.Precision.HIGHEST)
    li = lax.broadcasted_iota(jnp.int32, (NW, NW), 0)
    lj = lax.broadcasted_iota(jnp.int32, (NW, NW), 1)
    tri = (lj < li).astype(jnp.float32)
    bases = jnp.dot(tri, counts, preferred_element_type=jnp.float32,
                    precision=lax.Precision.HIGHEST)
    meta = jnp.concatenate(
        [bases.astype(jnp.int32), jnp.full((1, 1), tkey, jnp.int32)], axis=0)
    meta_ref[...] = jnp.broadcast_to(meta, (NW + 1, 128))


def _k3_body(key_hbm, f_hbm, meta_hbm, out_hbm,
             keyv, idxv, rowsv, outidxv, metav, sem, *, chunk, cout):
    wid = lax.axis_index("s") * NC + lax.axis_index("c")
    base_row = wid * chunk

    pltpu.sync_copy(meta_hbm.at[pl.ds(wid * 128, 16)], metav)
    base = jnp.max(metav[...])
    pltpu.sync_copy(meta_hbm.at[pl.ds(NW * 128, 16)], metav)
    tkey_vec = metav[...]
    pltpu.sync_copy(key_hbm.at[pl.ds(base_row, chunk)], keyv)

    def zbody(i, c):
        idxv[pl.ds(i * 16, 16)] = jnp.zeros((16,), jnp.int32)
        return c
    lax.fori_loop(0, (chunk + GBLK) // 16, zbody, jnp.int32(0))

    lane = lax.iota(jnp.int32, 16)

    def comp_body(i, off):
        vec = keyv[pl.ds(i * 16, 16)]
        m = vec > tkey_vec
        rowids = base_row + i * 16 + lane
        plsc.store_compressed(idxv.at[pl.ds(off, 16)], rowids, mask=m)
        return off + jnp.sum(m.astype(jnp.int32))

    cnt = lax.fori_loop(0, chunk // 16, comp_body, jnp.int32(0))

    c_eff = jnp.where(wid == NW - 1, jnp.maximum(cnt, TGT - base), cnt)

    nchunks = (c_eff + GBLK - 1) // GBLK

    def gw_body(j, c):
        jbase = j * GBLK

        def ob(z, c2):
            l = jbase + z * 16 + lane
            outidxv[pl.ds(z * 16, 16)] = jnp.where(
                l < c_eff, base + l, jnp.int32(TGT))
            return c2
        lax.fori_loop(0, GBLK // 16, ob, jnp.int32(0))
        pltpu.async_copy(f_hbm.at[idxv.at[pl.ds(jbase, GBLK)]],
                         rowsv, sem).wait()
        pltpu.async_copy(rowsv, out_hbm.at[outidxv], sem).wait()
        return c
    lax.fori_loop(0, nchunks, gw_body, jnp.int32(0))


def kernel(fea, W_up, b_up, W_cls, b_cls, target_points_num):
    n, cin = fea.shape
    cout = W_up.shape[1]
    blk = 2048

    f, key = pl.pallas_call(
        _k1_body,
        grid=(n // blk,),
        in_specs=[
            pl.BlockSpec((blk, cin), lambda i: (i, 0)),
            pl.BlockSpec((cin, cout), lambda i: (0, 0)),
            pl.BlockSpec((1, cout), lambda i: (0, 0)),
            pl.BlockSpec((cout, 1), lambda i: (0, 0)),
            pl.BlockSpec((1, 1), lambda i: (0, 0)),
        ],
        out_specs=[
            pl.BlockSpec((blk, cout), lambda i: (i, 0)),
            pl.BlockSpec((blk, 1), lambda i: (i, 0)),
        ],
        out_shape=[
            jax.ShapeDtypeStruct((n, cout), jnp.float32),
            jax.ShapeDtypeStruct((n, 1), jnp.int32),
        ],
    )(fea, W_up, b_up.reshape(1, cout), W_cls, b_cls.reshape(1, 1))

    rows = n // 128
    chunk = n // NW
    meta = pl.pallas_call(
        functools.partial(_k2_body, n=n, chunk_rows=rows // NW),
        out_shape=jax.ShapeDtypeStruct((NW + 1, 128), jnp.int32),
    )(key.reshape(rows, 128))

    mesh = plsc.VectorSubcoreMesh(core_axis_name="c", subcore_axis_name="s",
                                  num_cores=NC, num_subcores=NS)
    out_padded = pl.kernel(
        functools.partial(_k3_body, chunk=chunk, cout=cout),
        out_type=jax.ShapeDtypeStruct((TGT + GBLK, cout), jnp.float32),
        mesh=mesh,
        compiler_params=pltpu.CompilerParams(needs_layout_passes=False),
        scratch_types=[
            pltpu.VMEM((chunk,), jnp.int32),
            pltpu.VMEM((chunk + GBLK,), jnp.int32),
            pltpu.VMEM((GBLK, cout), jnp.float32),
            pltpu.VMEM((GBLK,), jnp.int32),
            pltpu.VMEM((16,), jnp.int32),
            pltpu.SemaphoreType.DMA,
        ],
    )(key.reshape(n), f, meta.reshape((NW + 1) * 128))

    return out_padded[:TGT]


# speedup vs baseline: 1.1774x; 1.0442x over previous
import functools

import jax
import jax.numpy as jnp
from jax import lax
from jax.experimental import pallas as pl
from jax.experimental.pallas import tpu as pltpu
from jax.experimental.pallas import tpu_sc as plsc

TGT = 131072
NC, NS = 2, 16
NW = NC * NS
GBLK = 128
SUPB = 3
SUP = SUPB * GBLK


def _key_from_f32(x_i32):
    return x_i32 ^ (lax.shift_right_arithmetic(x_i32, 31) & jnp.int32(0x7FFFFFFF))


def _k1_body(fea_ref, wup_ref, bup_ref, wcls_ref, bcls_ref, f_ref, key_ref):
    x = fea_ref[...]
    f = jnp.maximum(
        jnp.dot(x, wup_ref[...], preferred_element_type=jnp.float32)
        + bup_ref[...], 0.0)
    f_ref[...] = f
    pred = (jnp.dot(f, wcls_ref[...], preferred_element_type=jnp.float32)
            + bcls_ref[...])
    key_ref[...] = _key_from_f32(lax.bitcast_convert_type(pred, jnp.int32))


def _k2_body(key_ref, meta_ref, *, n, chunk_rows):
    key = key_ref[...]
    k0 = n - TGT

    def step(i, res):
        cand = res + lax.shift_left(jnp.int32(1), 31 - i)
        cnt = jnp.sum((key < cand).astype(jnp.int32))
        return jnp.where(cnt >= k0, res, cand)

    tkey = lax.fori_loop(0, 32, step, jnp.int32(-2147483647 - 1))

    keepf = (key > tkey).astype(jnp.float32)
    rowsum = jnp.sum(keepf, axis=1, keepdims=True)
    rows = key.shape[0]
    r_idx = lax.broadcasted_iota(jnp.int32, (NW, rows), 1)
    c_idx = lax.broadcasted_iota(jnp.int32, (NW, rows), 0)
    sel = (r_idx // chunk_rows == c_idx).astype(jnp.float32)
    counts = jnp.dot(sel, rowsum, preferred_element_type=jnp.float32,
                     precision=lax.Precision.HIGHEST)
    li = lax.broadcasted_iota(jnp.int32, (NW, NW), 0)
    lj = lax.broadcasted_iota(jnp.int32, (NW, NW), 1)
    tri = (lj < li).astype(jnp.float32)
    bases = jnp.dot(tri, counts, preferred_element_type=jnp.float32,
                    precision=lax.Precision.HIGHEST)
    meta = jnp.concatenate(
        [bases.astype(jnp.int32), jnp.full((1, 1), tkey, jnp.int32)], axis=0)
    meta_ref[...] = jnp.broadcast_to(meta, (NW + 1, 128))


def _k3_body(key_hbm, f_hbm, meta_hbm, out_hbm,
             keyv, idxv, s0, s1, oi0, oi1, outidxv, metav,
             gs0, gs1, ws0, ws1, *, chunk, cout):
    wid = lax.axis_index("s") * NC + lax.axis_index("c")
    base_row = wid * chunk

    pltpu.sync_copy(meta_hbm.at[pl.ds(wid * 128, 16)], metav)
    base = jnp.max(metav[...])
    pltpu.sync_copy(meta_hbm.at[pl.ds(NW * 128, 16)], metav)
    tkey_vec = metav[...]
    pltpu.sync_copy(key_hbm.at[pl.ds(base_row, chunk)], keyv)

    def zbody(i, c):
        idxv[pl.ds(i * 16, 16)] = jnp.zeros((16,), jnp.int32)
        return c
    lax.fori_loop(0, (chunk + 2 * GBLK) // 16, zbody, jnp.int32(0))

    lane = lax.iota(jnp.int32, 16)

    def comp_body(i, off):
        vec = keyv[pl.ds(i * 16, 16)]
        m = vec > tkey_vec
        rowids = base_row + i * 16 + lane
        plsc.store_compressed(idxv.at[pl.ds(off, 16)], rowids, mask=m)
        return off + jnp.sum(m.astype(jnp.int32))

    cnt = lax.fori_loop(0, chunk // 16, comp_body, jnp.int32(0))

    c_eff = jnp.where(wid == NW - 1, jnp.maximum(cnt, TGT - base), cnt)

    npairs = cnt // (2 * SUP)

    def fire(buf, sup, gsem):
        for k in range(SUPB):
            pltpu.async_copy(
                f_hbm.at[idxv.at[pl.ds(sup * SUP + k * GBLK, GBLK)]],
                buf.at[pl.ds(k * GBLK, GBLK)], gsem)

    def gdrain(buf, gsem):
        for k in range(SUPB):
            pltpu.make_async_copy(f_hbm.at[pl.ds(0, GBLK)],
                                  buf.at[pl.ds(k * GBLK, GBLK)], gsem).wait()

    def scat(buf, oi, sup, wsem):
        obase = base + sup * SUP
        for k in range(SUPB):
            def ob(z, c2):
                oi[k, pl.ds(z * 16, 16)] = obase + k * GBLK + z * 16 + lane
                return c2
            lax.fori_loop(0, GBLK // 16, ob, jnp.int32(0))
            pltpu.async_copy(buf.at[pl.ds(k * GBLK, GBLK)],
                             out_hbm.at[oi.at[k]], wsem)

    def wdrain(buf, wsem):
        for k in range(SUPB):
            pltpu.make_async_copy(f_hbm.at[pl.ds(0, GBLK)],
                                  buf.at[pl.ds(k * GBLK, GBLK)], wsem).wait()

    @pl.when(npairs > 0)
    def _():
        fire(s0, 0, gs0)
        fire(s1, 1, gs1)

    def pair_body(u, c):
        gdrain(s0, gs0)
        scat(s0, oi0, 2 * u, ws0)
        gdrain(s1, gs1)
        scat(s1, oi1, 2 * u + 1, ws1)

        @pl.when(u + 1 < npairs)
        def _():
            wdrain(s0, ws0)
            fire(s0, 2 * u + 2, gs0)
            wdrain(s1, ws1)
            fire(s1, 2 * u + 3, gs1)

        @pl.when(u + 1 == npairs)
        def _():
            wdrain(s0, ws0)
            wdrain(s1, ws1)
        return c
    lax.fori_loop(0, npairs, pair_body, jnp.int32(0))

    tail_start = npairs * 2 * SUP
    ntail = (c_eff - tail_start + GBLK - 1) // GBLK

    def tail_body(j, c):
        jbase = tail_start + j * GBLK

        def ob(z, c2):
            l = jbase + z * 16 + lane
            outidxv[pl.ds(z * 16, 16)] = jnp.where(
                l < c_eff, base + l, jnp.int32(TGT))
            return c2
        lax.fori_loop(0, GBLK // 16, ob, jnp.int32(0))
        pltpu.async_copy(f_hbm.at[idxv.at[pl.ds(jbase, GBLK)]],
                         s0.at[pl.ds(0, GBLK)], gs0).wait()
        pltpu.async_copy(s0.at[pl.ds(0, GBLK)], out_hbm.at[outidxv],
                         gs0).wait()
        return c
    lax.fori_loop(0, ntail, tail_body, jnp.int32(0))


def kernel(fea, W_up, b_up, W_cls, b_cls, target_points_num):
    n, cin = fea.shape
    cout = W_up.shape[1]
    blk = 2048

    f, key = pl.pallas_call(
        _k1_body,
        grid=(n // blk,),
        in_specs=[
            pl.BlockSpec((blk, cin), lambda i: (i, 0)),
            pl.BlockSpec((cin, cout), lambda i: (0, 0)),
            pl.BlockSpec((1, cout), lambda i: (0, 0)),
            pl.BlockSpec((cout, 1), lambda i: (0, 0)),
            pl.BlockSpec((1, 1), lambda i: (0, 0)),
        ],
        out_specs=[
            pl.BlockSpec((blk, cout), lambda i: (i, 0)),
            pl.BlockSpec((blk, 1), lambda i: (i, 0)),
        ],
        out_shape=[
            jax.ShapeDtypeStruct((n, cout), jnp.float32),
            jax.ShapeDtypeStruct((n, 1), jnp.int32),
        ],
    )(fea, W_up, b_up.reshape(1, cout), W_cls, b_cls.reshape(1, 1))

    rows = n // 128
    chunk = n // NW
    meta = pl.pallas_call(
        functools.partial(_k2_body, n=n, chunk_rows=rows // NW),
        out_shape=jax.ShapeDtypeStruct((NW + 1, 128), jnp.int32),
    )(key.reshape(rows, 128))

    mesh = plsc.VectorSubcoreMesh(core_axis_name="c", subcore_axis_name="s",
                                  num_cores=NC, num_subcores=NS)
    out_padded = pl.kernel(
        functools.partial(_k3_body, chunk=chunk, cout=cout),
        out_type=jax.ShapeDtypeStruct((TGT + GBLK, cout), jnp.float32),
        mesh=mesh,
        compiler_params=pltpu.CompilerParams(needs_layout_passes=False),
        scratch_types=[
            pltpu.VMEM((chunk,), jnp.int32),
            pltpu.VMEM((chunk + 2 * GBLK,), jnp.int32),
            pltpu.VMEM((SUP, cout), jnp.float32),
            pltpu.VMEM((SUP, cout), jnp.float32),
            pltpu.VMEM((SUPB, GBLK), jnp.int32),
            pltpu.VMEM((SUPB, GBLK), jnp.int32),
            pltpu.VMEM((GBLK,), jnp.int32),
            pltpu.VMEM((16,), jnp.int32),
            pltpu.SemaphoreType.DMA,
            pltpu.SemaphoreType.DMA,
            pltpu.SemaphoreType.DMA,
            pltpu.SemaphoreType.DMA,
        ],
    )(key.reshape(n), f, meta.reshape((NW + 1) * 128))

    return out_padded[:TGT]


# speedup vs baseline: 1.2914x; 1.0968x over previous
import functools

import jax
import jax.numpy as jnp
from jax import lax
from jax.experimental import pallas as pl
from jax.experimental.pallas import tpu as pltpu
from jax.experimental.pallas import tpu_sc as plsc

TGT = 131072
NC, NS = 2, 16
NW = NC * NS
GBLK = 128
SUPB = 3
SUP = SUPB * GBLK


def _key_from_f32(x_i32):
    return x_i32 ^ (lax.shift_right_arithmetic(x_i32, 31) & jnp.int32(0x7FFFFFFF))


def _k1_body(fea_ref, wup_ref, bup_ref, wcls_ref, bcls_ref, f_ref, key_ref):
    x = fea_ref[...]
    f = jnp.maximum(
        jnp.dot(x, wup_ref[...], preferred_element_type=jnp.float32)
        + bup_ref[...], 0.0)
    f_ref[...] = f
    pred = (jnp.dot(f, wcls_ref[...], preferred_element_type=jnp.float32)
            + bcls_ref[...])
    key_ref[...] = _key_from_f32(lax.bitcast_convert_type(pred, jnp.int32))


def _k2_body(key_ref, meta_ref, *, n, chunk_rows):
    key = key_ref[...]
    k0 = n - TGT

    def step(i, res):
        cand = res + lax.shift_left(jnp.int32(1), 31 - i)
        cnt = jnp.sum((key < cand).astype(jnp.int32))
        return jnp.where(cnt >= k0, res, cand)

    tkey = lax.fori_loop(0, 32, step, jnp.int32(-2147483647 - 1))

    keepf = (key > tkey).astype(jnp.float32)
    rowsum = jnp.sum(keepf, axis=1, keepdims=True)
    rows = key.shape[0]
    r_idx = lax.broadcasted_iota(jnp.int32, (NW, rows), 1)
    c_idx = lax.broadcasted_iota(jnp.int32, (NW, rows), 0)
    sel = (r_idx // chunk_rows == c_idx).astype(jnp.float32)
    counts = jnp.dot(sel, rowsum, preferred_element_type=jnp.float32,
                     precision=lax.Precision.HIGHEST)
    li = lax.broadcasted_iota(jnp.int32, (NW, NW), 0)
    lj = lax.broadcasted_iota(jnp.int32, (NW, NW), 1)
    tri = (lj < li).astype(jnp.float32)
    bases = jnp.dot(tri, counts, preferred_element_type=jnp.float32,
                    precision=lax.Precision.HIGHEST)
    meta = jnp.concatenate(
        [bases.astype(jnp.int32), jnp.full((1, 1), tkey, jnp.int32)], axis=0)
    meta_ref[...] = jnp.broadcast_to(meta, (NW + 1, 128))


def _k3_body(key_hbm, f_hbm, meta_hbm, out_hbm,
             keyv, idxv, s0, s1, oi0, oi1, outidxv, metav,
             gs0, gs1, ws0, ws1, *, chunk, cout):
    wid = lax.axis_index("s") * NC + lax.axis_index("c")
    base_row = wid * chunk

    pltpu.sync_copy(meta_hbm.at[pl.ds(wid * 128, 16)], metav)
    base = jnp.max(metav[...])
    pltpu.sync_copy(meta_hbm.at[pl.ds(NW * 128, 16)], metav)
    tkey_vec = metav[...]
    pltpu.sync_copy(key_hbm.at[pl.ds(base_row, chunk)], keyv)

    def zbody(i, c):
        idxv[pl.ds(i * 16, 16)] = jnp.zeros((16,), jnp.int32)
        return c
    lax.fori_loop(0, (chunk + 2 * GBLK) // 16, zbody, jnp.int32(0))

    lane = lax.iota(jnp.int32, 16)

    def comp_body(i, off):
        vec = keyv[pl.ds(i * 16, 16)]
        m = vec > tkey_vec
        rowids = base_row + i * 16 + lane
        plsc.store_compressed(idxv.at[pl.ds(off, 16)], rowids, mask=m)
        return off + jnp.sum(m.astype(jnp.int32))

    cnt = lax.fori_loop(0, chunk // 16, comp_body, jnp.int32(0))

    c_eff = jnp.where(wid == NW - 1, jnp.maximum(cnt, TGT - base), cnt)

    npairs = cnt // (2 * SUP)

    def fire(buf, sup, gsem):
        for k in range(SUPB):
            pltpu.async_copy(
                f_hbm.at[idxv.at[pl.ds(sup * SUP + k * GBLK, GBLK)]],
                buf.at[pl.ds(k * GBLK, GBLK)], gsem)

    def gdrain(buf, gsem):
        for k in range(SUPB):
            pltpu.make_async_copy(f_hbm.at[pl.ds(0, GBLK)],
                                  buf.at[pl.ds(k * GBLK, GBLK)], gsem).wait()

    def scat(buf, oi, sup, wsem):
        obase = base + sup * SUP
        for k in range(SUPB):
            def ob(z, c2):
                oi[k, pl.ds(z * 16, 16)] = obase + k * GBLK + z * 16 + lane
                return c2
            lax.fori_loop(0, GBLK // 16, ob, jnp.int32(0))
            pltpu.async_copy(buf.at[pl.ds(k * GBLK, GBLK)],
                             out_hbm.at[oi.at[k]], wsem)

    def wdrain(buf, wsem):
        for k in range(SUPB):
            pltpu.make_async_copy(f_hbm.at[pl.ds(0, GBLK)],
                                  buf.at[pl.ds(k * GBLK, GBLK)], wsem).wait()

    @pl.when(npairs > 0)
    def _():
        fire(s0, 0, gs0)
        fire(s1, 1, gs1)

    def pair_body(u, c):
        gdrain(s0, gs0)
        scat(s0, oi0, 2 * u, ws0)
        gdrain(s1, gs1)
        scat(s1, oi1, 2 * u + 1, ws1)

        @pl.when(u + 1 < npairs)
        def _():
            wdrain(s0, ws0)
            fire(s0, 2 * u + 2, gs0)
            wdrain(s1, ws1)
            fire(s1, 2 * u + 3, gs1)

        @pl.when(u + 1 == npairs)
        def _():
            wdrain(s0, ws0)
            wdrain(s1, ws1)
        return c
    lax.fori_loop(0, npairs, pair_body, jnp.int32(0))

    tail_start = npairs * 2 * SUP
    ntail = (c_eff - tail_start + GBLK - 1) // GBLK

    def tail_body(j, c):
        jbase = tail_start + j * GBLK

        def ob(z, c2):
            l = jbase + z * 16 + lane
            outidxv[pl.ds(z * 16, 16)] = jnp.where(
                l < c_eff, base + l, jnp.int32(TGT))
            return c2
        lax.fori_loop(0, GBLK // 16, ob, jnp.int32(0))
        pltpu.async_copy(f_hbm.at[idxv.at[pl.ds(jbase, GBLK)]],
                         s0.at[pl.ds(0, GBLK)], gs0).wait()
        pltpu.async_copy(s0.at[pl.ds(0, GBLK)], out_hbm.at[outidxv],
                         gs0).wait()
        return c
    lax.fori_loop(0, ntail, tail_body, jnp.int32(0))


def kernel(fea, W_up, b_up, W_cls, b_cls, target_points_num):
    n, cin = fea.shape
    cout = W_up.shape[1]
    blk = 4096

    f, key = pl.pallas_call(
        _k1_body,
        grid=(n // blk,),
        in_specs=[
            pl.BlockSpec((blk, cin), lambda i: (i, 0)),
            pl.BlockSpec((cin, cout), lambda i: (0, 0)),
            pl.BlockSpec((1, cout), lambda i: (0, 0)),
            pl.BlockSpec((cout, 1), lambda i: (0, 0)),
            pl.BlockSpec((1, 1), lambda i: (0, 0)),
        ],
        out_specs=[
            pl.BlockSpec((blk, cout), lambda i: (i, 0)),
            pl.BlockSpec((blk, 1), lambda i: (i, 0)),
        ],
        out_shape=[
            jax.ShapeDtypeStruct((n, cout), jnp.float32),
            jax.ShapeDtypeStruct((n, 1), jnp.int32),
        ],
    )(fea, W_up, b_up.reshape(1, cout), W_cls, b_cls.reshape(1, 1))

    rows = n // 128
    chunk = n // NW
    meta = pl.pallas_call(
        functools.partial(_k2_body, n=n, chunk_rows=rows // NW),
        out_shape=jax.ShapeDtypeStruct((NW + 1, 128), jnp.int32),
    )(key.reshape(rows, 128))

    mesh = plsc.VectorSubcoreMesh(core_axis_name="c", subcore_axis_name="s",
                                  num_cores=NC, num_subcores=NS)
    out_padded = pl.kernel(
        functools.partial(_k3_body, chunk=chunk, cout=cout),
        out_type=jax.ShapeDtypeStruct((TGT + GBLK, cout), jnp.float32),
        mesh=mesh,
        compiler_params=pltpu.CompilerParams(needs_layout_passes=False),
        scratch_types=[
            pltpu.VMEM((chunk,), jnp.int32),
            pltpu.VMEM((chunk + 2 * GBLK,), jnp.int32),
            pltpu.VMEM((SUP, cout), jnp.float32),
            pltpu.VMEM((SUP, cout), jnp.float32),
            pltpu.VMEM((SUPB, GBLK), jnp.int32),
            pltpu.VMEM((SUPB, GBLK), jnp.int32),
            pltpu.VMEM((GBLK,), jnp.int32),
            pltpu.VMEM((16,), jnp.int32),
            pltpu.SemaphoreType.DMA,
            pltpu.SemaphoreType.DMA,
            pltpu.SemaphoreType.DMA,
            pltpu.SemaphoreType.DMA,
        ],
    )(key.reshape(n), f, meta.reshape((NW + 1) * 128))

    return out_padded[:TGT]


# speedup vs baseline: 1.3211x; 1.0231x over previous
import functools

import jax
import jax.numpy as jnp
from jax import lax
from jax.experimental import pallas as pl
from jax.experimental.pallas import tpu as pltpu
from jax.experimental.pallas import tpu_sc as plsc

TGT = 131072
NC, NS = 2, 16
NW = NC * NS
GBLK = 128
SUPB = 3
SUP = SUPB * GBLK


def _key_from_f32(x_i32):
    return x_i32 ^ (lax.shift_right_arithmetic(x_i32, 31) & jnp.int32(0x7FFFFFFF))


def _k1_body(fea_ref, wup_ref, bup_ref, wcls_ref, bcls_ref, f_ref, key_ref):
    x = fea_ref[...]
    f = jnp.maximum(
        jnp.dot(x, wup_ref[...], preferred_element_type=jnp.float32)
        + bup_ref[...], 0.0)
    f_ref[...] = f
    pred = (jnp.dot(f, wcls_ref[...], preferred_element_type=jnp.float32)
            + bcls_ref[...])
    key_ref[...] = _key_from_f32(lax.bitcast_convert_type(pred, jnp.int32))


def _k2_body(key_ref, meta_ref, *, n, chunk_rows):
    key = key_ref[...]
    k0 = n - TGT

    def step(i, res):
        cand = res + lax.shift_left(jnp.int32(1), 31 - i)
        cnt = jnp.sum((key < cand).astype(jnp.int32))
        return jnp.where(cnt >= k0, res, cand)

    tkey = lax.fori_loop(0, 32, step, jnp.int32(-2147483647 - 1))

    keepf = (key > tkey).astype(jnp.float32)
    rowsum = jnp.sum(keepf, axis=1, keepdims=True)
    rows = key.shape[0]
    r_idx = lax.broadcasted_iota(jnp.int32, (NW, rows), 1)
    c_idx = lax.broadcasted_iota(jnp.int32, (NW, rows), 0)
    sel = (r_idx // chunk_rows == c_idx).astype(jnp.float32)
    counts = jnp.dot(sel, rowsum, preferred_element_type=jnp.float32,
                     precision=lax.Precision.HIGHEST)
    li = lax.broadcasted_iota(jnp.int32, (NW, NW), 0)
    lj = lax.broadcasted_iota(jnp.int32, (NW, NW), 1)
    tri = (lj < li).astype(jnp.float32)
    bases = jnp.dot(tri, counts, preferred_element_type=jnp.float32,
                    precision=lax.Precision.HIGHEST)
    meta = jnp.concatenate(
        [bases.astype(jnp.int32), jnp.full((1, 1), tkey, jnp.int32)], axis=0)
    meta_ref[...] = jnp.broadcast_to(meta, (NW + 1, 128))


def _k3_body(key_hbm, f_hbm, meta_hbm, out_hbm,
             keyv, idxv, s0, s1, oi0, oi1, outidxv, metav,
             gs0, gs1, ws0, ws1, *, chunk, cout):
    wid = lax.axis_index("s") * NC + lax.axis_index("c")
    base_row = wid * chunk

    pltpu.sync_copy(meta_hbm.at[pl.ds(wid * 128, 16)], metav)
    base = jnp.max(metav[...])
    pltpu.sync_copy(meta_hbm.at[pl.ds(NW * 128, 16)], metav)
    tkey_vec = metav[...]
    pltpu.sync_copy(key_hbm.at[pl.ds(base_row, chunk)], keyv)

    def zbody(i, c):
        idxv[pl.ds(i * 16, 16)] = jnp.zeros((16,), jnp.int32)
        return c
    lax.fori_loop(0, (chunk + 2 * GBLK) // 16, zbody, jnp.int32(0))

    lane = lax.iota(jnp.int32, 16)

    def comp_body(i, off):
        vec = keyv[pl.ds(i * 16, 16)]
        m = vec > tkey_vec
        rowids = base_row + i * 16 + lane
        plsc.store_compressed(idxv.at[pl.ds(off, 16)], rowids, mask=m)
        return off + jnp.sum(m.astype(jnp.int32))

    cnt = lax.fori_loop(0, chunk // 16, comp_body, jnp.int32(0))

    c_eff = jnp.where(wid == NW - 1, jnp.maximum(cnt, TGT - base), cnt)

    npairs = cnt // (2 * SUP)

    def fire(buf, sup, gsem):
        for k in range(SUPB):
            pltpu.async_copy(
                f_hbm.at[idxv.at[pl.ds(sup * SUP + k * GBLK, GBLK)]],
                buf.at[pl.ds(k * GBLK, GBLK)], gsem)

    def gdrain(buf, gsem):
        for k in range(SUPB):
            pltpu.make_async_copy(f_hbm.at[pl.ds(0, GBLK)],
                                  buf.at[pl.ds(k * GBLK, GBLK)], gsem).wait()

    def scat(buf, oi, sup, wsem):
        obase = base + sup * SUP
        for k in range(SUPB):
            def ob(z, c2):
                oi[k, pl.ds(z * 16, 16)] = obase + k * GBLK + z * 16 + lane
                return c2
            lax.fori_loop(0, GBLK // 16, ob, jnp.int32(0))
            pltpu.async_copy(buf.at[pl.ds(k * GBLK, GBLK)],
                             out_hbm.at[oi.at[k]], wsem)

    def wdrain(buf, wsem):
        for k in range(SUPB):
            pltpu.make_async_copy(f_hbm.at[pl.ds(0, GBLK)],
                                  buf.at[pl.ds(k * GBLK, GBLK)], wsem).wait()

    @pl.when(npairs > 0)
    def _():
        fire(s0, 0, gs0)
        fire(s1, 1, gs1)

    def pair_body(u, c):
        gdrain(s0, gs0)
        scat(s0, oi0, 2 * u, ws0)
        gdrain(s1, gs1)
        scat(s1, oi1, 2 * u + 1, ws1)

        @pl.when(u + 1 < npairs)
        def _():
            wdrain(s0, ws0)
            fire(s0, 2 * u + 2, gs0)
            wdrain(s1, ws1)
            fire(s1, 2 * u + 3, gs1)

        @pl.when(u + 1 == npairs)
        def _():
            wdrain(s0, ws0)
            wdrain(s1, ws1)
        return c
    lax.fori_loop(0, npairs, pair_body, jnp.int32(0))

    tail_start = npairs * 2 * SUP
    ntail = (c_eff - tail_start + GBLK - 1) // GBLK

    def tail_body(j, c):
        jbase = tail_start + j * GBLK

        def ob(z, c2):
            l = jbase + z * 16 + lane
            outidxv[pl.ds(z * 16, 16)] = jnp.where(
                l < c_eff, base + l, jnp.int32(TGT))
            return c2
        lax.fori_loop(0, GBLK // 16, ob, jnp.int32(0))
        pltpu.async_copy(f_hbm.at[idxv.at[pl.ds(jbase, GBLK)]],
                         s0.at[pl.ds(0, GBLK)], gs0).wait()
        pltpu.async_copy(s0.at[pl.ds(0, GBLK)], out_hbm.at[outidxv],
                         gs0).wait()
        return c
    lax.fori_loop(0, ntail, tail_body, jnp.int32(0))


def kernel(fea, W_up, b_up, W_cls, b_cls, target_points_num):
    n, cin = fea.shape
    cout = W_up.shape[1]
    blk = 8192

    f, key = pl.pallas_call(
        _k1_body,
        grid=(n // blk,),
        in_specs=[
            pl.BlockSpec((blk, cin), lambda i: (i, 0)),
            pl.BlockSpec((cin, cout), lambda i: (0, 0)),
            pl.BlockSpec((1, cout), lambda i: (0, 0)),
            pl.BlockSpec((cout, 1), lambda i: (0, 0)),
            pl.BlockSpec((1, 1), lambda i: (0, 0)),
        ],
        out_specs=[
            pl.BlockSpec((blk, cout), lambda i: (i, 0)),
            pl.BlockSpec((blk, 1), lambda i: (i, 0)),
        ],
        out_shape=[
            jax.ShapeDtypeStruct((n, cout), jnp.float32),
            jax.ShapeDtypeStruct((n, 1), jnp.int32),
        ],
    )(fea, W_up, b_up.reshape(1, cout), W_cls, b_cls.reshape(1, 1))

    rows = n // 128
    chunk = n // NW
    meta = pl.pallas_call(
        functools.partial(_k2_body, n=n, chunk_rows=rows // NW),
        out_shape=jax.ShapeDtypeStruct((NW + 1, 128), jnp.int32),
    )(key.reshape(rows, 128))

    mesh = plsc.VectorSubcoreMesh(core_axis_name="c", subcore_axis_name="s",
                                  num_cores=NC, num_subcores=NS)
    out_padded = pl.kernel(
        functools.partial(_k3_body, chunk=chunk, cout=cout),
        out_type=jax.ShapeDtypeStruct((TGT + GBLK, cout), jnp.float32),
        mesh=mesh,
        compiler_params=pltpu.CompilerParams(needs_layout_passes=False),
        scratch_types=[
            pltpu.VMEM((chunk,), jnp.int32),
            pltpu.VMEM((chunk + 2 * GBLK,), jnp.int32),
            pltpu.VMEM((SUP, cout), jnp.float32),
            pltpu.VMEM((SUP, cout), jnp.float32),
            pltpu.VMEM((SUPB, GBLK), jnp.int32),
            pltpu.VMEM((SUPB, GBLK), jnp.int32),
            pltpu.VMEM((GBLK,), jnp.int32),
            pltpu.VMEM((16,), jnp.int32),
            pltpu.SemaphoreType.DMA,
            pltpu.SemaphoreType.DMA,
            pltpu.SemaphoreType.DMA,
            pltpu.SemaphoreType.DMA,
        ],
    )(key.reshape(n), f, meta.reshape((NW + 1) * 128))

    return out_padded[:TGT]


# speedup vs baseline: 1.3286x; 1.0056x over previous
import functools

import jax
import jax.numpy as jnp
from jax import lax
from jax.experimental import pallas as pl
from jax.experimental.pallas import tpu as pltpu
from jax.experimental.pallas import tpu_sc as plsc

TGT = 131072
NC, NS = 2, 16
NW = NC * NS
GBLK = 128
SUPB = 3
SUP = SUPB * GBLK


def _key_from_f32(x_i32):
    return x_i32 ^ (lax.shift_right_arithmetic(x_i32, 31) & jnp.int32(0x7FFFFFFF))


def _k1_body(fea_ref, wup_ref, bup_ref, wcls_ref, bcls_ref, f_ref, key_ref):
    x = fea_ref[...]
    f = jnp.maximum(
        jnp.dot(x, wup_ref[...], preferred_element_type=jnp.float32)
        + bup_ref[...], 0.0)
    f_ref[...] = f
    pred = (jnp.dot(f, wcls_ref[...], preferred_element_type=jnp.float32)
            + bcls_ref[...])
    key_ref[...] = _key_from_f32(lax.bitcast_convert_type(pred, jnp.int32))


def _k2_body(key_ref, meta_ref, *, n, chunk_rows):
    key = key_ref[...]
    k0 = n - TGT

    def step(i, res):
        cand = res + lax.shift_left(jnp.int32(1), 31 - i)
        cnt = jnp.sum((key < cand).astype(jnp.int32))
        return jnp.where(cnt >= k0, res, cand)

    tkey = lax.fori_loop(0, 32, step, jnp.int32(-2147483647 - 1))

    keepf = (key > tkey).astype(jnp.float32)
    rowsum = jnp.sum(keepf, axis=1, keepdims=True)
    rows = key.shape[0]
    r_idx = lax.broadcasted_iota(jnp.int32, (NW, rows), 1)
    c_idx = lax.broadcasted_iota(jnp.int32, (NW, rows), 0)
    sel = (r_idx // chunk_rows == c_idx).astype(jnp.float32)
    counts = jnp.dot(sel, rowsum, preferred_element_type=jnp.float32,
                     precision=lax.Precision.HIGHEST)
    li = lax.broadcasted_iota(jnp.int32, (NW, NW), 0)
    lj = lax.broadcasted_iota(jnp.int32, (NW, NW), 1)
    tri = (lj < li).astype(jnp.float32)
    bases = jnp.dot(tri, counts, preferred_element_type=jnp.float32,
                    precision=lax.Precision.HIGHEST)
    meta = jnp.concatenate(
        [bases.astype(jnp.int32), jnp.full((1, 1), tkey, jnp.int32)], axis=0)
    meta_ref[...] = jnp.broadcast_to(meta, (NW + 1, 128))


def _k3_body(key_hbm, f_hbm, meta_hbm, out_hbm,
             keyv, idxv, s0, s1, oi0, oi1, outidxv, metav,
             gs0, gs1, ws0, ws1, *, chunk, cout):
    wid = lax.axis_index("s") * NC + lax.axis_index("c")
    base_row = wid * chunk

    pltpu.sync_copy(meta_hbm.at[pl.ds(wid * 128, 16)], metav)
    base = jnp.max(metav[...])
    pltpu.sync_copy(meta_hbm.at[pl.ds(NW * 128, 16)], metav)
    tkey_vec = metav[...]
    pltpu.sync_copy(key_hbm.at[pl.ds(base_row, chunk)], keyv)

    def zbody(i, c):
        idxv[pl.ds(i * 16, 16)] = jnp.zeros((16,), jnp.int32)
        return c
    lax.fori_loop(0, (chunk + 2 * GBLK) // 16, zbody, jnp.int32(0))

    lane = lax.iota(jnp.int32, 16)

    def comp_body(i, off):
        vec = keyv[pl.ds(i * 16, 16)]
        m = vec > tkey_vec
        rowids = base_row + i * 16 + lane
        plsc.store_compressed(idxv.at[pl.ds(off, 16)], rowids, mask=m)
        return off + jnp.sum(m.astype(jnp.int32))

    cnt = lax.fori_loop(0, chunk // 16, comp_body, jnp.int32(0))

    c_eff = jnp.where(wid == NW - 1, jnp.maximum(cnt, TGT - base), cnt)

    npairs = cnt // (2 * SUP)

    def fire(buf, sup, gsem):
        for k in range(SUPB):
            pltpu.async_copy(
                f_hbm.at[idxv.at[pl.ds(sup * SUP + k * GBLK, GBLK)]],
                buf.at[pl.ds(k * GBLK, GBLK)], gsem)

    def gdrain(buf, gsem):
        for k in range(SUPB):
            pltpu.make_async_copy(f_hbm.at[pl.ds(0, GBLK)],
                                  buf.at[pl.ds(k * GBLK, GBLK)], gsem).wait()

    def scat(buf, oi, sup, wsem):
        obase = base + sup * SUP
        for k in range(SUPB):
            def ob(z, c2):
                oi[k, pl.ds(z * 16, 16)] = obase + k * GBLK + z * 16 + lane
                return c2
            lax.fori_loop(0, GBLK // 16, ob, jnp.int32(0))
            pltpu.async_copy(buf.at[pl.ds(k * GBLK, GBLK)],
                             out_hbm.at[oi.at[k]], wsem)

    def wdrain(buf, wsem):
        for k in range(SUPB):
            pltpu.make_async_copy(f_hbm.at[pl.ds(0, GBLK)],
                                  buf.at[pl.ds(k * GBLK, GBLK)], wsem).wait()

    @pl.when(npairs > 0)
    def _():
        fire(s0, 0, gs0)
        fire(s1, 1, gs1)

    def pair_body(u, c):
        gdrain(s0, gs0)
        scat(s0, oi0, 2 * u, ws0)
        gdrain(s1, gs1)
        scat(s1, oi1, 2 * u + 1, ws1)

        @pl.when(u + 1 < npairs)
        def _():
            wdrain(s0, ws0)
            fire(s0, 2 * u + 2, gs0)
            wdrain(s1, ws1)
            fire(s1, 2 * u + 3, gs1)

        @pl.when(u + 1 == npairs)
        def _():
            wdrain(s0, ws0)
            wdrain(s1, ws1)
        return c
    lax.fori_loop(0, npairs, pair_body, jnp.int32(0))

    tail_start = npairs * 2 * SUP
    ntail = (c_eff - tail_start + GBLK - 1) // GBLK

    def tail_body(j, c):
        jbase = tail_start + j * GBLK

        def ob(z, c2):
            l = jbase + z * 16 + lane
            outidxv[pl.ds(z * 16, 16)] = jnp.where(
                l < c_eff, base + l, jnp.int32(TGT))
            return c2
        lax.fori_loop(0, GBLK // 16, ob, jnp.int32(0))
        pltpu.async_copy(f_hbm.at[idxv.at[pl.ds(jbase, GBLK)]],
                         s0.at[pl.ds(0, GBLK)], gs0).wait()
        pltpu.async_copy(s0.at[pl.ds(0, GBLK)], out_hbm.at[outidxv],
                         gs0).wait()
        return c
    lax.fori_loop(0, ntail, tail_body, jnp.int32(0))


def kernel(fea, W_up, b_up, W_cls, b_cls, target_points_num):
    n, cin = fea.shape
    cout = W_up.shape[1]
    blk = 16384

    f, key = pl.pallas_call(
        _k1_body,
        grid=(n // blk,),
        in_specs=[
            pl.BlockSpec((blk, cin), lambda i: (i, 0)),
            pl.BlockSpec((cin, cout), lambda i: (0, 0)),
            pl.BlockSpec((1, cout), lambda i: (0, 0)),
            pl.BlockSpec((cout, 1), lambda i: (0, 0)),
            pl.BlockSpec((1, 1), lambda i: (0, 0)),
        ],
        out_specs=[
            pl.BlockSpec((blk, cout), lambda i: (i, 0)),
            pl.BlockSpec((blk, 1), lambda i: (i, 0)),
        ],
        out_shape=[
            jax.ShapeDtypeStruct((n, cout), jnp.float32),
            jax.ShapeDtypeStruct((n, 1), jnp.int32),
        ],
    )(fea, W_up, b_up.reshape(1, cout), W_cls, b_cls.reshape(1, 1))

    rows = n // 128
    chunk = n // NW
    meta = pl.pallas_call(
        functools.partial(_k2_body, n=n, chunk_rows=rows // NW),
        out_shape=jax.ShapeDtypeStruct((NW + 1, 128), jnp.int32),
    )(key.reshape(rows, 128))

    mesh = plsc.VectorSubcoreMesh(core_axis_name="c", subcore_axis_name="s",
                                  num_cores=NC, num_subcores=NS)
    out_padded = pl.kernel(
        functools.partial(_k3_body, chunk=chunk, cout=cout),
        out_type=jax.ShapeDtypeStruct((TGT + GBLK, cout), jnp.float32),
        mesh=mesh,
        compiler_params=pltpu.CompilerParams(needs_layout_passes=False),
        scratch_types=[
            pltpu.VMEM((chunk,), jnp.int32),
            pltpu.VMEM((chunk + 2 * GBLK,), jnp.int32),
            pltpu.VMEM((SUP, cout), jnp.float32),
            pltpu.VMEM((SUP, cout), jnp.float32),
            pltpu.VMEM((SUPB, GBLK), jnp.int32),
            pltpu.VMEM((SUPB, GBLK), jnp.int32),
            pltpu.VMEM((GBLK,), jnp.int32),
            pltpu.VMEM((16,), jnp.int32),
            pltpu.SemaphoreType.DMA,
            pltpu.SemaphoreType.DMA,
            pltpu.SemaphoreType.DMA,
            pltpu.SemaphoreType.DMA,
        ],
    )(key.reshape(n), f, meta.reshape((NW + 1) * 128))

    return out_padded[:TGT]


# speedup vs baseline: 1.3438x; 1.0114x over previous
import functools

import jax
import jax.numpy as jnp
from jax import lax
from jax.experimental import pallas as pl
from jax.experimental.pallas import tpu as pltpu
from jax.experimental.pallas import tpu_sc as plsc

TGT = 131072
NC, NS = 2, 16
NW = NC * NS
GBLK = 128
SUPB = 2
SUP = SUPB * GBLK


def _key_from_f32(x_i32):
    return x_i32 ^ (lax.shift_right_arithmetic(x_i32, 31) & jnp.int32(0x7FFFFFFF))


def _k1_body(fea_ref, wup_ref, bup_ref, wcls_ref, bcls_ref, f_ref, key_ref):
    x = fea_ref[...]
    f = jnp.maximum(
        jnp.dot(x, wup_ref[...], preferred_element_type=jnp.float32)
        + bup_ref[...], 0.0)
    f_ref[...] = f
    pred = (jnp.dot(f, wcls_ref[...], preferred_element_type=jnp.float32)
            + bcls_ref[...])
    key_ref[...] = _key_from_f32(lax.bitcast_convert_type(pred, jnp.int32))


def _k2_body(key_ref, meta_ref, *, n, chunk_rows):
    key = key_ref[...]
    k0 = n - TGT

    def step(i, res):
        cand = res + lax.shift_left(jnp.int32(1), 31 - i)
        cnt = jnp.sum((key < cand).astype(jnp.int32))
        return jnp.where(cnt >= k0, res, cand)

    tkey = lax.fori_loop(0, 32, step, jnp.int32(-2147483647 - 1))

    keepf = (key > tkey).astype(jnp.float32)
    rowsum = jnp.sum(keepf, axis=1, keepdims=True)
    rows = key.shape[0]
    r_idx = lax.broadcasted_iota(jnp.int32, (NW, rows), 1)
    c_idx = lax.broadcasted_iota(jnp.int32, (NW, rows), 0)
    sel = (r_idx // chunk_rows == c_idx).astype(jnp.float32)
    counts = jnp.dot(sel, rowsum, preferred_element_type=jnp.float32,
                     precision=lax.Precision.HIGHEST)
    li = lax.broadcasted_iota(jnp.int32, (NW, NW), 0)
    lj = lax.broadcasted_iota(jnp.int32, (NW, NW), 1)
    tri = (lj < li).astype(jnp.float32)
    bases = jnp.dot(tri, counts, preferred_element_type=jnp.float32,
                    precision=lax.Precision.HIGHEST)
    meta = jnp.concatenate(
        [bases.astype(jnp.int32), jnp.full((1, 1), tkey, jnp.int32)], axis=0)
    meta_ref[...] = jnp.broadcast_to(meta, (NW + 1, 128))


def _k3_body(key_hbm, f_hbm, meta_hbm, out_hbm,
             keyv, idxv, s0, s1, s2, oi0, oi1, oi2, outidxv, metav,
             gs0, gs1, gs2, ws0, ws1, ws2, *, chunk, cout):
    wid = lax.axis_index("s") * NC + lax.axis_index("c")
    base_row = wid * chunk

    pltpu.sync_copy(meta_hbm.at[pl.ds(wid * 128, 16)], metav)
    base = jnp.max(metav[...])
    pltpu.sync_copy(meta_hbm.at[pl.ds(NW * 128, 16)], metav)
    tkey_vec = metav[...]
    pltpu.sync_copy(key_hbm.at[pl.ds(base_row, chunk)], keyv)

    def zbody(i, c):
        idxv[pl.ds(i * 16, 16)] = jnp.zeros((16,), jnp.int32)
        return c
    lax.fori_loop(0, (chunk + 2 * GBLK) // 16, zbody, jnp.int32(0))

    lane = lax.iota(jnp.int32, 16)

    def comp_body(i, off):
        vec = keyv[pl.ds(i * 16, 16)]
        m = vec > tkey_vec
        rowids = base_row + i * 16 + lane
        plsc.store_compressed(idxv.at[pl.ds(off, 16)], rowids, mask=m)
        return off + jnp.sum(m.astype(jnp.int32))

    cnt = lax.fori_loop(0, chunk // 16, comp_body, jnp.int32(0))

    c_eff = jnp.where(wid == NW - 1, jnp.maximum(cnt, TGT - base), cnt)

    niter = cnt // (3 * SUP)

    def fire(buf, sup, gsem):
        for k in range(SUPB):
            pltpu.async_copy(
                f_hbm.at[idxv.at[pl.ds(sup * SUP + k * GBLK, GBLK)]],
                buf.at[pl.ds(k * GBLK, GBLK)], gsem)

    def gdrain(buf, gsem):
        for k in range(SUPB):
            pltpu.make_async_copy(f_hbm.at[pl.ds(0, GBLK)],
                                  buf.at[pl.ds(k * GBLK, GBLK)], gsem).wait()

    def scat(buf, oi, sup, wsem):
        obase = base + sup * SUP
        for k in range(SUPB):
            def ob(z, c2):
                oi[k, pl.ds(z * 16, 16)] = obase + k * GBLK + z * 16 + lane
                return c2
            lax.fori_loop(0, GBLK // 16, ob, jnp.int32(0))
            pltpu.async_copy(buf.at[pl.ds(k * GBLK, GBLK)],
                             out_hbm.at[oi.at[k]], wsem)

    def wdrain(buf, wsem):
        for k in range(SUPB):
            pltpu.make_async_copy(f_hbm.at[pl.ds(0, GBLK)],
                                  buf.at[pl.ds(k * GBLK, GBLK)], wsem).wait()

    @pl.when(niter > 0)
    def _():
        fire(s0, 0, gs0)
        fire(s1, 1, gs1)

    def ring_body(u, c):
        gdrain(s0, gs0)
        scat(s0, oi0, 3 * u, ws0)

        @pl.when(u > 0)
        def _():
            wdrain(s2, ws2)
        fire(s2, 3 * u + 2, gs2)

        gdrain(s1, gs1)
        scat(s1, oi1, 3 * u + 1, ws1)

        wdrain(s0, ws0)

        @pl.when(u + 1 < niter)
        def _():
            fire(s0, 3 * u + 3, gs0)

        gdrain(s2, gs2)
        scat(s2, oi2, 3 * u + 2, ws2)

        wdrain(s1, ws1)

        @pl.when(u + 1 < niter)
        def _():
            fire(s1, 3 * u + 4, gs1)
        return c
    lax.fori_loop(0, niter, ring_body, jnp.int32(0))

    @pl.when(niter > 0)
    def _():
        wdrain(s2, ws2)

    tail_start = niter * 3 * SUP
    ntail = (c_eff - tail_start + GBLK - 1) // GBLK

    def tail_body(j, c):
        jbase = tail_start + j * GBLK

        def ob(z, c2):
            l = jbase + z * 16 + lane
            outidxv[pl.ds(z * 16, 16)] = jnp.where(
                l < c_eff, base + l, jnp.int32(TGT))
            return c2
        lax.fori_loop(0, GBLK // 16, ob, jnp.int32(0))
        pltpu.async_copy(f_hbm.at[idxv.at[pl.ds(jbase, GBLK)]],
                         s0.at[pl.ds(0, GBLK)], gs0).wait()
        pltpu.async_copy(s0.at[pl.ds(0, GBLK)], out_hbm.at[outidxv],
                         gs0).wait()
        return c
    lax.fori_loop(0, ntail, tail_body, jnp.int32(0))


def kernel(fea, W_up, b_up, W_cls, b_cls, target_points_num):
    n, cin = fea.shape
    cout = W_up.shape[1]
    blk = 16384

    f, key = pl.pallas_call(
        _k1_body,
        grid=(n // blk,),
        in_specs=[
            pl.BlockSpec((blk, cin), lambda i: (i, 0)),
            pl.BlockSpec((cin, cout), lambda i: (0, 0)),
            pl.BlockSpec((1, cout), lambda i: (0, 0)),
            pl.BlockSpec((cout, 1), lambda i: (0, 0)),
            pl.BlockSpec((1, 1), lambda i: (0, 0)),
        ],
        out_specs=[
            pl.BlockSpec((blk, cout), lambda i: (i, 0)),
            pl.BlockSpec((blk, 1), lambda i: (i, 0)),
        ],
        out_shape=[
            jax.ShapeDtypeStruct((n, cout), jnp.float32),
            jax.ShapeDtypeStruct((n, 1), jnp.int32),
        ],
    )(fea, W_up, b_up.reshape(1, cout), W_cls, b_cls.reshape(1, 1))

    rows = n // 128
    chunk = n // NW
    meta = pl.pallas_call(
        functools.partial(_k2_body, n=n, chunk_rows=rows // NW),
        out_shape=jax.ShapeDtypeStruct((NW + 1, 128), jnp.int32),
    )(key.reshape(rows, 128))

    mesh = plsc.VectorSubcoreMesh(core_axis_name="c", subcore_axis_name="s",
                                  num_cores=NC, num_subcores=NS)
    out_padded = pl.kernel(
        functools.partial(_k3_body, chunk=chunk, cout=cout),
        out_type=jax.ShapeDtypeStruct((TGT + GBLK, cout), jnp.float32),
        mesh=mesh,
        compiler_params=pltpu.CompilerParams(needs_layout_passes=False),
        scratch_types=[
            pltpu.VMEM((chunk,), jnp.int32),
            pltpu.VMEM((chunk + 2 * GBLK,), jnp.int32),
            pltpu.VMEM((SUP, cout), jnp.float32),
            pltpu.VMEM((SUP, cout), jnp.float32),
            pltpu.VMEM((SUP, cout), jnp.float32),
            pltpu.VMEM((SUPB, GBLK), jnp.int32),
            pltpu.VMEM((SUPB, GBLK), jnp.int32),
            pltpu.VMEM((SUPB, GBLK), jnp.int32),
            pltpu.VMEM((GBLK,), jnp.int32),
            pltpu.VMEM((16,), jnp.int32),
            pltpu.SemaphoreType.DMA,
            pltpu.SemaphoreType.DMA,
            pltpu.SemaphoreType.DMA,
            pltpu.SemaphoreType.DMA,
            pltpu.SemaphoreType.DMA,
            pltpu.SemaphoreType.DMA,
        ],
    )(key.reshape(n), f, meta.reshape((NW + 1) * 128))

    return out_padded[:TGT]
